# Initial kernel scaffold; baseline (speedup 1.0000x reference)
import jax
import jax.numpy as jnp
from jax.experimental import pallas as pl


def kernel(x, wih0, wih_rest, whh, b, wfc, bfc):
    raise NotImplementedError("write your pallas kernel here")


# speedup vs baseline: 1.4235x; 1.4235x over previous
import jax
import jax.numpy as jnp
from jax.experimental import pallas as pl
from jax.experimental.pallas import tpu as pltpu


def _lstm_act(gates, c):
    H = c.shape[-1]
    sif = jax.nn.sigmoid(gates[:, :2 * H])
    i_g = sif[:, :H]
    f_g = sif[:, H:]
    g_g = jnp.tanh(gates[:, 2 * H:3 * H])
    o_g = jax.nn.sigmoid(gates[:, 3 * H:])
    c_new = f_g * c + i_g * g_g
    h_new = o_g * jnp.tanh(c_new)
    return h_new, c_new


def _wavefront_kernel(x_ref, wih0_ref, w1_ref, w2_ref, b1_ref, b2_ref,
                      wfc_ref, bfc_ref, out_ref,
                      proj_ref, h1_ref, c1_ref, h2_ref, c2_ref):
    ti = pl.program_id(1)
    Tt, Bt, I = x_ref.shape
    cd = w1_ref.dtype

    @pl.when(ti == 0)
    def _():
        h1_ref[...] = jnp.zeros_like(h1_ref)
        c1_ref[...] = jnp.zeros_like(c1_ref)
        h2_ref[...] = jnp.zeros_like(h2_ref)
        c2_ref[...] = jnp.zeros_like(c2_ref)

    x2d = x_ref[...].reshape(Tt * Bt, I)
    proj_ref[...] = (
        jnp.dot(x2d, wih0_ref[...], preferred_element_type=jnp.float32)
        + b1_ref[...]
    ).reshape(Tt, Bt, proj_ref.shape[-1])

    h1 = h1_ref[...]
    c1 = c1_ref[...]
    h2 = h2_ref[...]
    c2 = c2_ref[...]
    b2 = b2_ref[...]

    for i in range(Tt):
        lhs2 = jnp.concatenate([h1, h2], axis=1)
        g1 = proj_ref[i] + jnp.dot(h1, w1_ref[...],
                                   preferred_element_type=jnp.float32)
        g2 = jnp.dot(lhs2, w2_ref[...],
                     preferred_element_type=jnp.float32) + b2
        h1n, c1n = _lstm_act(g1, c1)
        h2n, c2n = _lstm_act(g2, c2)
        h1n = h1n.astype(cd)
        h2n = h2n.astype(cd)
        if i == 0:
            valid = ti > 0
            h2n = jnp.where(valid, h2n, h2)
            c2n = jnp.where(valid, c2n, c2)
        h1, c1, h2, c2 = h1n, c1n, h2n, c2n

    h1_ref[...] = h1
    c1_ref[...] = c1
    h2_ref[...] = h2
    c2_ref[...] = c2

    @pl.when(ti == pl.num_programs(1) - 1)
    def _():
        lhs2 = jnp.concatenate([h1, h2], axis=1)
        g2 = jnp.dot(lhs2, w2_ref[...],
                     preferred_element_type=jnp.float32) + b2
        h2f, _ = _lstm_act(g2, c2)
        out_ref[...] = (
            jnp.dot(h2f.astype(cd), wfc_ref[...],
                    preferred_element_type=jnp.float32)
            + bfc_ref[...]
        )


def kernel(x, wih0, wih_rest, whh, b, wfc, bfc):
    B, T, I = x.shape
    H = whh.shape[1]
    G = 4 * H
    O = wfc.shape[1]
    cd = jnp.bfloat16

    b_tile = 128 if B % 128 == 0 else B
    t_tile = 32 if T % 32 == 0 else T

    x_tm = jnp.transpose(x, (1, 0, 2)).astype(cd)
    wih0c = wih0.astype(cd)
    w1 = whh[0].astype(cd)
    w2 = jnp.concatenate([wih_rest[0], whh[1]], axis=0).astype(cd)
    b1 = b[0]
    b2 = b[1]
    wfcc = wfc.astype(cd)

    grid = (B // b_tile, T // t_tile)

    out = pl.pallas_call(
        _wavefront_kernel,
        out_shape=jax.ShapeDtypeStruct((B, O), jnp.float32),
        grid_spec=pltpu.PrefetchScalarGridSpec(
            num_scalar_prefetch=0,
            grid=grid,
            in_specs=[
                pl.BlockSpec((t_tile, b_tile, I), lambda bi, ti: (ti, bi, 0)),
                pl.BlockSpec((I, G), lambda bi, ti: (0, 0)),
                pl.BlockSpec((H, G), lambda bi, ti: (0, 0)),
                pl.BlockSpec((2 * H, G), lambda bi, ti: (0, 0)),
                pl.BlockSpec((1, G), lambda bi, ti: (0, 0)),
                pl.BlockSpec((1, G), lambda bi, ti: (0, 0)),
                pl.BlockSpec((H, O), lambda bi, ti: (0, 0)),
                pl.BlockSpec((1, O), lambda bi, ti: (0, 0)),
            ],
            out_specs=pl.BlockSpec((b_tile, O), lambda bi, ti: (bi, 0)),
            scratch_shapes=[
                pltpu.VMEM((t_tile, b_tile, G), jnp.float32),
                pltpu.VMEM((b_tile, H), cd),
                pltpu.VMEM((b_tile, H), jnp.float32),
                pltpu.VMEM((b_tile, H), cd),
                pltpu.VMEM((b_tile, H), jnp.float32),
            ],
        ),
        compiler_params=pltpu.CompilerParams(
            dimension_semantics=("parallel", "arbitrary")),
    )(x_tm, wih0c, w1, w2, b1, b2, wfcc, bfc)

    return out


# speedup vs baseline: 1.5940x; 1.1198x over previous
import jax
import jax.numpy as jnp
from jax.experimental import pallas as pl
from jax.experimental.pallas import tpu as pltpu


def _sigmoid(x):
    return jnp.tanh(x * 0.5) * 0.5 + 0.5


def _lstm_act(gates, c):
    H = c.shape[-1]
    sif = _sigmoid(gates[:, :2 * H])
    i_g = sif[:, :H]
    f_g = sif[:, H:]
    g_g = jnp.tanh(gates[:, 2 * H:3 * H])
    o_g = _sigmoid(gates[:, 3 * H:])
    c_new = f_g * c + i_g * g_g
    h_new = o_g * jnp.tanh(c_new)
    return h_new, c_new


def _wavefront_kernel(x_ref, wih0_ref, w1_ref, w2_ref, b1_ref, b2_ref,
                      wfc_ref, bfc_ref, out_ref,
                      proj_ref, h1_ref, c1_ref, h2_ref, c2_ref):
    ti = pl.program_id(1)
    Tt, Bt, I = x_ref.shape
    cd = w1_ref.dtype

    @pl.when(ti == 0)
    def _():
        h1_ref[...] = jnp.zeros_like(h1_ref)
        c1_ref[...] = jnp.zeros_like(c1_ref)
        h2_ref[...] = jnp.zeros_like(h2_ref)
        c2_ref[...] = jnp.zeros_like(c2_ref)

    x2d = x_ref[...].reshape(Tt * Bt, I)
    proj_ref[...] = (
        jnp.dot(x2d, wih0_ref[...], preferred_element_type=jnp.float32)
        + b1_ref[...]
    ).reshape(Tt, Bt, proj_ref.shape[-1])

    h1 = h1_ref[...]
    c1 = c1_ref[...]
    h2 = h2_ref[...]
    c2 = c2_ref[...]
    b2 = b2_ref[...]

    for i in range(Tt):
        lhs2 = jnp.concatenate([h1, h2], axis=1)
        g1 = proj_ref[i] + jnp.dot(h1, w1_ref[...],
                                   preferred_element_type=jnp.float32)
        g2 = jnp.dot(lhs2, w2_ref[...],
                     preferred_element_type=jnp.float32) + b2
        h1n, c1n = _lstm_act(g1, c1)
        h2n, c2n = _lstm_act(g2, c2)
        h1n = h1n.astype(cd)
        h2n = h2n.astype(cd)
        if i == 0:
            valid = ti > 0
            h2n = jnp.where(valid, h2n, h2)
            c2n = jnp.where(valid, c2n, c2)
        h1, c1, h2, c2 = h1n, c1n, h2n, c2n

    h1_ref[...] = h1
    c1_ref[...] = c1
    h2_ref[...] = h2
    c2_ref[...] = c2

    @pl.when(ti == pl.num_programs(1) - 1)
    def _():
        lhs2 = jnp.concatenate([h1, h2], axis=1)
        g2 = jnp.dot(lhs2, w2_ref[...],
                     preferred_element_type=jnp.float32) + b2
        h2f, _ = _lstm_act(g2, c2)
        out_ref[...] = (
            jnp.dot(h2f.astype(cd), wfc_ref[...],
                    preferred_element_type=jnp.float32)
            + bfc_ref[...]
        )


def kernel(x, wih0, wih_rest, whh, b, wfc, bfc):
    B, T, I = x.shape
    H = whh.shape[1]
    G = 4 * H
    O = wfc.shape[1]
    cd = jnp.bfloat16

    b_tile = 128 if B % 128 == 0 else B
    t_tile = 32 if T % 32 == 0 else T

    x_tm = jnp.transpose(x, (1, 0, 2)).astype(cd)
    wih0c = wih0.astype(cd)
    w1 = whh[0].astype(cd)
    w2 = jnp.concatenate([wih_rest[0], whh[1]], axis=0).astype(cd)
    b1 = b[0]
    b2 = b[1]
    wfcc = wfc.astype(cd)

    grid = (B // b_tile, T // t_tile)

    out = pl.pallas_call(
        _wavefront_kernel,
        out_shape=jax.ShapeDtypeStruct((B, O), jnp.float32),
        grid_spec=pltpu.PrefetchScalarGridSpec(
            num_scalar_prefetch=0,
            grid=grid,
            in_specs=[
                pl.BlockSpec((t_tile, b_tile, I), lambda bi, ti: (ti, bi, 0)),
                pl.BlockSpec((I, G), lambda bi, ti: (0, 0)),
                pl.BlockSpec((H, G), lambda bi, ti: (0, 0)),
                pl.BlockSpec((2 * H, G), lambda bi, ti: (0, 0)),
                pl.BlockSpec((1, G), lambda bi, ti: (0, 0)),
                pl.BlockSpec((1, G), lambda bi, ti: (0, 0)),
                pl.BlockSpec((H, O), lambda bi, ti: (0, 0)),
                pl.BlockSpec((1, O), lambda bi, ti: (0, 0)),
            ],
            out_specs=pl.BlockSpec((b_tile, O), lambda bi, ti: (bi, 0)),
            scratch_shapes=[
                pltpu.VMEM((t_tile, b_tile, G), jnp.float32),
                pltpu.VMEM((b_tile, H), cd),
                pltpu.VMEM((b_tile, H), jnp.float32),
                pltpu.VMEM((b_tile, H), cd),
                pltpu.VMEM((b_tile, H), jnp.float32),
            ],
        ),
        compiler_params=pltpu.CompilerParams(
            dimension_semantics=("parallel", "arbitrary")),
    )(x_tm, wih0c, w1, w2, b1, b2, wfcc, bfc)

    return out


# speedup vs baseline: 1.5972x; 1.0020x over previous
import jax
import jax.numpy as jnp
from jax.experimental import pallas as pl
from jax.experimental.pallas import tpu as pltpu


def _sigmoid(x):
    return jnp.tanh(x * 0.5) * 0.5 + 0.5


def _lstm_act(gates, c):
    H = c.shape[-1]
    sif = _sigmoid(gates[:, :2 * H])
    i_g = sif[:, :H]
    f_g = sif[:, H:]
    g_g = jnp.tanh(gates[:, 2 * H:3 * H])
    o_g = _sigmoid(gates[:, 3 * H:])
    c_new = f_g * c + i_g * g_g
    h_new = o_g * jnp.tanh(c_new)
    return h_new, c_new


def _wavefront_kernel(x_ref, wih0_ref, w1_ref, w2_ref, b1_ref, b2_ref,
                      wfc_ref, bfc_ref, out_ref,
                      proj_ref, h1_ref, c1_ref, h2_ref, c2_ref):
    ti = pl.program_id(1)
    Tt, Bt, I = x_ref.shape
    cd = w1_ref.dtype

    @pl.when(ti == 0)
    def _():
        h1_ref[...] = jnp.zeros_like(h1_ref)
        c1_ref[...] = jnp.zeros_like(c1_ref)
        h2_ref[...] = jnp.zeros_like(h2_ref)
        c2_ref[...] = jnp.zeros_like(c2_ref)

    x2d = x_ref[...].reshape(Tt * Bt, I)
    proj_ref[...] = (
        jnp.dot(x2d, wih0_ref[...], preferred_element_type=jnp.float32)
        + b1_ref[...]
    ).reshape(Tt, Bt, proj_ref.shape[-1])

    h1 = h1_ref[...]
    c1 = c1_ref[...]
    h2 = h2_ref[...]
    c2 = c2_ref[...]
    b2 = b2_ref[...]

    for i in range(Tt):
        lhs2 = jnp.concatenate([h1, h2], axis=1)
        g1 = proj_ref[i] + jnp.dot(h1, w1_ref[...],
                                   preferred_element_type=jnp.float32)
        g2 = jnp.dot(lhs2, w2_ref[...],
                     preferred_element_type=jnp.float32) + b2
        h1n, c1n = _lstm_act(g1, c1)
        h2n, c2n = _lstm_act(g2, c2)
        h1n = h1n.astype(cd)
        h2n = h2n.astype(cd)
        if i == 0:
            valid = ti > 0
            h2n = jnp.where(valid, h2n, h2)
            c2n = jnp.where(valid, c2n, c2)
        h1, c1, h2, c2 = h1n, c1n, h2n, c2n

    h1_ref[...] = h1
    c1_ref[...] = c1
    h2_ref[...] = h2
    c2_ref[...] = c2

    @pl.when(ti == pl.num_programs(1) - 1)
    def _():
        lhs2 = jnp.concatenate([h1, h2], axis=1)
        g2 = jnp.dot(lhs2, w2_ref[...],
                     preferred_element_type=jnp.float32) + b2
        h2f, _ = _lstm_act(g2, c2)
        out_ref[...] = (
            jnp.dot(h2f.astype(cd), wfc_ref[...],
                    preferred_element_type=jnp.float32)
            + bfc_ref[...]
        )


def kernel(x, wih0, wih_rest, whh, b, wfc, bfc):
    B, T, I = x.shape
    H = whh.shape[1]
    G = 4 * H
    O = wfc.shape[1]
    cd = jnp.bfloat16

    b_tile = 128 if B % 128 == 0 else B
    t_tile = 64 if T % 64 == 0 else T

    x_tm = jnp.transpose(x, (1, 0, 2)).astype(cd)
    wih0c = wih0.astype(cd)
    w1 = whh[0].astype(cd)
    w2 = jnp.concatenate([wih_rest[0], whh[1]], axis=0).astype(cd)
    b1 = b[0]
    b2 = b[1]
    wfcc = wfc.astype(cd)

    grid = (B // b_tile, T // t_tile)

    out = pl.pallas_call(
        _wavefront_kernel,
        out_shape=jax.ShapeDtypeStruct((B, O), jnp.float32),
        grid_spec=pltpu.PrefetchScalarGridSpec(
            num_scalar_prefetch=0,
            grid=grid,
            in_specs=[
                pl.BlockSpec((t_tile, b_tile, I), lambda bi, ti: (ti, bi, 0)),
                pl.BlockSpec((I, G), lambda bi, ti: (0, 0)),
                pl.BlockSpec((H, G), lambda bi, ti: (0, 0)),
                pl.BlockSpec((2 * H, G), lambda bi, ti: (0, 0)),
                pl.BlockSpec((1, G), lambda bi, ti: (0, 0)),
                pl.BlockSpec((1, G), lambda bi, ti: (0, 0)),
                pl.BlockSpec((H, O), lambda bi, ti: (0, 0)),
                pl.BlockSpec((1, O), lambda bi, ti: (0, 0)),
            ],
            out_specs=pl.BlockSpec((b_tile, O), lambda bi, ti: (bi, 0)),
            scratch_shapes=[
                pltpu.VMEM((t_tile, b_tile, G), jnp.float32),
                pltpu.VMEM((b_tile, H), cd),
                pltpu.VMEM((b_tile, H), jnp.float32),
                pltpu.VMEM((b_tile, H), cd),
                pltpu.VMEM((b_tile, H), jnp.float32),
            ],
        ),
        compiler_params=pltpu.CompilerParams(
            dimension_semantics=("parallel", "arbitrary")),
    )(x_tm, wih0c, w1, w2, b1, b2, wfcc, bfc)

    return out


# speedup vs baseline: 1.8601x; 1.1646x over previous
import jax
import jax.numpy as jnp
from jax.experimental import pallas as pl
from jax.experimental.pallas import tpu as pltpu


def _sigmoid(x):
    return jnp.tanh(x * 0.5) * 0.5 + 0.5


def _lstm_act(gates, c):
    H = c.shape[-1]
    sif = _sigmoid(gates[:, :2 * H])
    i_g = sif[:, :H]
    f_g = sif[:, H:]
    g_g = jnp.tanh(gates[:, 2 * H:3 * H])
    o_g = _sigmoid(gates[:, 3 * H:])
    c_new = f_g * c + i_g * g_g
    h_new = o_g * jnp.tanh(c_new)
    return h_new, c_new


def _wavefront_kernel(x_ref, w1_ref, w2_ref, b1_ref, b2_ref,
                      wfc_ref, bfc_ref, out_ref,
                      h1_ref, c1_ref, h2_ref, c2_ref):
    ti = pl.program_id(1)
    Tt, Bt, I = x_ref.shape
    cd = w1_ref.dtype

    @pl.when(ti == 0)
    def _():
        h1_ref[...] = jnp.zeros_like(h1_ref)
        c1_ref[...] = jnp.zeros_like(c1_ref)
        h2_ref[...] = jnp.zeros_like(h2_ref)
        c2_ref[...] = jnp.zeros_like(c2_ref)

    h1 = h1_ref[...]
    c1 = c1_ref[...]
    h2 = h2_ref[...]
    c2 = c2_ref[...]
    b1 = b1_ref[...]
    b2 = b2_ref[...]

    for i in range(Tt):
        lhs1 = jnp.concatenate([h1, x_ref[i]], axis=1)
        lhs2 = jnp.concatenate([h1, h2], axis=1)
        g1 = jnp.dot(lhs1, w1_ref[...],
                     preferred_element_type=jnp.float32) + b1
        g2 = jnp.dot(lhs2, w2_ref[...],
                     preferred_element_type=jnp.float32) + b2
        h1n, c1n = _lstm_act(g1, c1)
        h2n, c2n = _lstm_act(g2, c2)
        h1n = h1n.astype(cd)
        h2n = h2n.astype(cd)
        if i == 0:
            valid = ti > 0
            h2n = jnp.where(valid, h2n, h2)
            c2n = jnp.where(valid, c2n, c2)
        h1, c1, h2, c2 = h1n, c1n, h2n, c2n

    h1_ref[...] = h1
    c1_ref[...] = c1
    h2_ref[...] = h2
    c2_ref[...] = c2

    @pl.when(ti == pl.num_programs(1) - 1)
    def _():
        lhs2 = jnp.concatenate([h1, h2], axis=1)
        g2 = jnp.dot(lhs2, w2_ref[...],
                     preferred_element_type=jnp.float32) + b2
        h2f, _ = _lstm_act(g2, c2)
        out_ref[...] = (
            jnp.dot(h2f.astype(cd), wfc_ref[...],
                    preferred_element_type=jnp.float32)
            + bfc_ref[...]
        )


def kernel(x, wih0, wih_rest, whh, b, wfc, bfc):
    B, T, I = x.shape
    H = whh.shape[1]
    G = 4 * H
    O = wfc.shape[1]
    cd = jnp.bfloat16

    b_tile = 128 if B % 128 == 0 else B
    t_tile = 64 if T % 64 == 0 else T

    x_tm = jnp.transpose(x, (1, 0, 2)).astype(cd)
    w1 = jnp.concatenate([whh[0], wih0], axis=0).astype(cd)
    w2 = jnp.concatenate([wih_rest[0], whh[1]], axis=0).astype(cd)
    b1 = b[0]
    b2 = b[1]
    wfcc = wfc.astype(cd)

    grid = (B // b_tile, T // t_tile)

    out = pl.pallas_call(
        _wavefront_kernel,
        out_shape=jax.ShapeDtypeStruct((B, O), jnp.float32),
        grid_spec=pltpu.PrefetchScalarGridSpec(
            num_scalar_prefetch=0,
            grid=grid,
            in_specs=[
                pl.BlockSpec((t_tile, b_tile, I), lambda bi, ti: (ti, bi, 0)),
                pl.BlockSpec((H + I, G), lambda bi, ti: (0, 0)),
                pl.BlockSpec((2 * H, G), lambda bi, ti: (0, 0)),
                pl.BlockSpec((1, G), lambda bi, ti: (0, 0)),
                pl.BlockSpec((1, G), lambda bi, ti: (0, 0)),
                pl.BlockSpec((H, O), lambda bi, ti: (0, 0)),
                pl.BlockSpec((1, O), lambda bi, ti: (0, 0)),
            ],
            out_specs=pl.BlockSpec((b_tile, O), lambda bi, ti: (bi, 0)),
            scratch_shapes=[
                pltpu.VMEM((b_tile, H), cd),
                pltpu.VMEM((b_tile, H), jnp.float32),
                pltpu.VMEM((b_tile, H), cd),
                pltpu.VMEM((b_tile, H), jnp.float32),
            ],
        ),
        compiler_params=pltpu.CompilerParams(
            dimension_semantics=("parallel", "arbitrary")),
    )(x_tm, w1, w2, b1, b2, wfcc, bfc)

    return out
